# baseline (speedup 1.0000x reference)
import functools

import jax
import jax.numpy as jnp
from jax import lax
from jax.experimental import pallas as pl
from jax.experimental.pallas import tpu as pltpu
from jax.experimental.pallas import tpu_sc as plsc

G = 128
SUPER = 2
RC = 32
NTILE = 16
BR = 1024


def _edge_split(e2, etot, c, s):
  base_c = jnp.where(c == 0, 0, e2)
  cnt = jnp.where(c == 0, e2, etot - e2)
  per = ((cnt + NTILE - 1) // NTILE + 7) // 8 * 8
  my0 = base_c + s * per
  mycnt = jnp.clip(cnt - s * per, 0, per)
  return my0, mycnt


@functools.lru_cache(maxsize=None)
def _make_spmm(npad, H, e2, etot, per0, per1):
  PT = npad // NTILE
  NQ = H // 16
  CH = SUPER * G
  mesh = plsc.VectorSubcoreMesh(core_axis_name="c", subcore_axis_name="s")

  scratch = [
      pltpu.VMEM_SHARED((npad, H), jnp.float32),
      pltpu.VMEM((2 * 3 * G,), jnp.int32),
      pltpu.VMEM((2, G), jnp.int32),
      pltpu.VMEM((2, G, H), jnp.float32),
      pltpu.VMEM((RC, H), jnp.float32),
      pltpu.SemaphoreType.DMA,
      pltpu.SemaphoreType.DMA,
      pltpu.SemaphoreType.DMA,
      pltpu.SemaphoreType.DMA,
      pltpu.SemaphoreType.DMA,
      pltpu.SemaphoreType.DMA,
      pltpu.SemaphoreType.DMA,
  ]

  def body(packed_h, u_h, p0_h, p1_h,
           acc, pbuf, rbuf, gbuf, zbuf, semz, semi0, semi1, semg0, semg1,
           sems0, sems1):
    c = lax.axis_index("c")
    s = lax.axis_index("s")
    semi = [semi0, semi1]
    semg = [semg0, semg1]
    sems = [sems0, sems1]
    per = jnp.where(c == 0, per0, per1)
    my0 = jnp.where(c == 0, 0, e2) + s * per
    cnt = jnp.where(c == 0, e2, etot - e2)
    mycnt = jnp.clip(cnt - s * per, 0, per)
    nch = (mycnt + G - 1) // G

    def zrow(i, _):
      for q in range(NQ):
        zbuf[i, pl.ds(q * 16, 16)] = jnp.zeros((16,), jnp.float32)
      return 0
    lax.fori_loop(0, RC, zrow, 0)

    r0 = s * PT

    def zfire(i, _):
      ri = pl.multiple_of(r0 + i * RC, 8)
      pltpu.async_copy(zbuf, acc.at[pl.ds(ri, RC)], semz)
      return 0
    lax.fori_loop(0, PT // RC, zfire, 0)

    def zdrain(i, _):
      ri = pl.multiple_of(r0 + i * RC, 8)
      pltpu.make_async_copy(zbuf, acc.at[pl.ds(ri, RC)], semz).wait()
      return 0
    lax.fori_loop(0, PT // RC, zdrain, 0)
    plsc.subcore_barrier()

    lanes0 = lax.iota(jnp.int32, 16)

    def pslice(b, part, n=3 * G):
      return pbuf.at[pl.ds(b * 3 * G + part * G, n)]

    def idx_src(i):
      off3 = pl.multiple_of((my0 // G + i) * 3 * G, 8)
      return packed_h.at[pl.ds(off3, 3 * G)]

    def idx_issue(i, b):
      pltpu.async_copy(idx_src(i), pslice(b, 0), semi[b])

    def idx_wait(i, b):
      pltpu.make_async_copy(idx_src(i), pslice(b, 0), semi[b]).wait()

    def gather_issue(b):
      pltpu.async_copy(u_h.at[pslice(b, 0, G)], gbuf.at[b], semg[b])

    def gather_wait(b):
      pltpu.make_async_copy(u_h.at[pslice(b, 0, G)], gbuf.at[b],
                            semg[b]).wait()

    def process(i, b):
      rem = mycnt - i * G

      def grp(j, _):
        vi = pbuf[pl.ds(b * 3 * G + G + j * 16, 16)]
        vv = lax.bitcast_convert_type(vi, jnp.float32)
        vv = jnp.where(j * 16 + lanes0 < rem, vv, 0.0)
        for l in range(16):
          val = vv[l]
          g = j * 16 + l
          for qq in range(NQ):
            sl = pl.ds(qq * 16, 16)
            gbuf[b, g, sl] = gbuf[b, g, sl] * val
        return 0
      lax.fori_loop(0, G // 16, grp, 0)
      for j in range(8):
        sl = pl.ds(j * 16, 16)
        rbuf[b, sl] = pbuf[pl.ds(b * 3 * G + 2 * G + j * 16, 16)]
      pltpu.async_copy(gbuf.at[b], acc.at[rbuf.at[b]], sems[b], add=True)

    def scat_wait(b):
      pltpu.make_async_copy(gbuf.at[b], acc.at[rbuf.at[b]],
                            sems[b]).wait()

    @pl.when(nch > 0)
    def _():
      pltpu.sync_copy(idx_src(0), pslice(0, 0))
      gather_issue(0)

    @pl.when(nch > 1)
    def _():
      idx_issue(1, 1)

    def pair(p, _):
      i0, i1, i2, i3 = 2 * p, 2 * p + 1, 2 * p + 2, 2 * p + 3

      @pl.when(i1 < nch)
      def _():
        idx_wait(i1, 1)
        gather_issue(1)

      gather_wait(0)
      process(i0, 0)

      @pl.when(i1 < nch)
      def _():
        gather_wait(1)
        process(i1, 1)

      scat_wait(0)

      @pl.when(i2 < nch)
      def _():
        idx_issue(i2, 0)
        idx_wait(i2, 0)
        gather_issue(0)

      @pl.when(i1 < nch)
      def _():
        scat_wait(1)

      @pl.when(i3 < nch)
      def _():
        idx_issue(i3, 1)
      return 0
    lax.fori_loop(0, (nch + 1) // 2, pair, 0)
    plsc.subcore_barrier()

    @pl.when(jnp.logical_and(s == 0, c == 0))
    def _():
      pltpu.sync_copy(acc, p0_h)

    @pl.when(jnp.logical_and(s == 0, c == 1))
    def _():
      pltpu.sync_copy(acc, p1_h)

  out_type = [jax.ShapeDtypeStruct((npad, H), jnp.float32)] * 2
  return pl.kernel(body, out_type=out_type, mesh=mesh,
                   scratch_types=scratch, name="cheb_spmm")


def _edge_pad_len(E, e2):
  per0 = -(-(-(-e2 // NTILE)) // G) * G
  per1 = -(-(-(-(E - e2) // NTILE)) // G) * G
  end0 = NTILE * per0 + SUPER * G
  end1 = e2 + NTILE * per1 + SUPER * G
  n = max(end0, end1, E)
  return -(-n // G) * G, per0, per1


@functools.lru_cache(maxsize=None)
def _make_deg(npad, ep, e2, etot):
  mesh = plsc.VectorSubcoreMesh(core_axis_name="c", subcore_axis_name="s")
  PT = npad // NTILE

  scratch = [
      pltpu.VMEM_SHARED((npad,), jnp.float32),
      pltpu.VMEM((G,), jnp.int32),
      pltpu.VMEM((G,), jnp.float32),
      pltpu.VMEM((RC * 16,), jnp.float32),
  ]

  def body(rows_h, d0_h, d1_h, accd, rbuf, obuf, zbuf):
    c = lax.axis_index("c")
    s = lax.axis_index("s")
    my0, mycnt = _edge_split(e2, etot, c, s)
    nch = (mycnt + G - 1) // G

    def zrow(i, _):
      zbuf[pl.ds(i * 16, 16)] = jnp.zeros((16,), jnp.float32)
      return 0
    lax.fori_loop(0, RC, zrow, 0)
    r0 = s * PT

    def zc(i, _):
      ri = pl.multiple_of(r0 + i * RC * 16, 8)
      pltpu.sync_copy(zbuf, accd.at[pl.ds(ri, RC * 16)])
      return 0
    lax.fori_loop(0, PT // (RC * 16), zc, 0)
    plsc.subcore_barrier()

    lanes0 = lax.iota(jnp.int32, 16)

    def chunk(i, _):
      off = pl.multiple_of(my0 + i * G, 8)
      pltpu.sync_copy(rows_h.at[pl.ds(off, G)], rbuf)
      rem = mycnt - i * G
      for j in range(G // 16):
        ones = jnp.where(j * 16 + lanes0 < rem, 1.0, 0.0)
        obuf[pl.ds(j * 16, 16)] = ones
      pltpu.sync_copy(obuf, accd.at[rbuf], add=True)
      return 0
    lax.fori_loop(0, nch, chunk, 0)
    plsc.subcore_barrier()

    @pl.when(jnp.logical_and(s == 0, c == 0))
    def _():
      pltpu.sync_copy(accd, d0_h)

    @pl.when(jnp.logical_and(s == 0, c == 1))
    def _():
      pltpu.sync_copy(accd, d1_h)

  out_type = [jax.ShapeDtypeStruct((npad,), jnp.float32)] * 2
  return pl.kernel(body, out_type=out_type, mesh=mesh,
                   scratch_types=scratch, name="deg_count")


def _degmax_tc(d0, d1):
  def body(a_ref, b_ref, o_ref):
    m = jnp.max(a_ref[...] + b_ref[...])
    o_ref[...] = jnp.full((8, 128), m, jnp.float32)
  return pl.pallas_call(
      body, out_shape=jax.ShapeDtypeStruct((8, 128), jnp.float32))(d0, d1)


@functools.lru_cache(maxsize=None)
def _make_merge(npad, H, aux):
  grid = npad // BR

  def body(*refs):
    if aux:
      coef, p0, p1, u, p, o, xa, ta, t_out, o_out = refs
    else:
      coef, p0, p1, u, p, o, t_out, o_out = refs
    a_, b_, d_ = coef[0], coef[1], coef[2]
    f0, f1, f2 = coef[3], coef[4], coef[5]
    uv = u[...]
    t = a_ * (p0[...] + p1[...]) + b_ * uv + d_ * p[...]
    oo = f0 * o[...] + f1 * uv + f2 * t
    if aux:
      oo = oo + coef[6] * xa[...] + coef[7] * ta[...]
    t_out[...] = t
    o_out[...] = oo

  narr = 7 if aux else 5
  bspec = pl.BlockSpec((BR, H), lambda i: (i, 0))
  return pl.pallas_call(
      body,
      grid=(grid,),
      in_specs=[pl.BlockSpec(memory_space=pltpu.SMEM)] + [bspec] * narr,
      out_specs=[bspec, bspec],
      out_shape=[jax.ShapeDtypeStruct((npad, H), jnp.float32)] * 2,
  )


def kernel(x, edge_index, edge_vals, poly_logits, hp_alpha):
  N, H = x.shape
  E = edge_index.shape[1]
  K = poly_logits.shape[0] - 1
  npad = -(-N // BR) * BR

  rows = edge_index[0]
  cols = edge_index[1]
  e2 = -(-(-(-E // 2)) // G) * G
  nbg, per0, per1 = _edge_pad_len(E, e2)
  zpad = jnp.zeros((nbg - E,), jnp.int32)
  rows_p = jnp.concatenate([rows, zpad])
  cols_p = jnp.concatenate([cols, zpad])
  vbits = lax.bitcast_convert_type(edge_vals, jnp.int32)
  vals_p = jnp.concatenate([vbits, zpad])
  nb = nbg // G
  packed = jnp.stack([cols_p.reshape(nb, G), vals_p.reshape(nb, G),
                      rows_p.reshape(nb, G)], axis=1).reshape(-1)
  ep = nbg

  xp = jnp.concatenate([x, jnp.zeros((npad - N, H), jnp.float32)])

  deg = _make_deg(npad, ep, e2, E)
  d0, d1 = deg(rows_p)
  dm = _degmax_tc(d0.reshape(npad // 128, 128), d1.reshape(npad // 128, 128))
  lam = 2.0 * dm[0, 0]
  c2 = 2.0 / lam

  w = jax.nn.softmax(poly_logits)
  alpha = hp_alpha.astype(jnp.float32)

  def mk(a, b, d, f0, f1, f2, f3=0.0, f4=0.0):
    return jnp.stack([jnp.asarray(v, jnp.float32) * jnp.ones((), jnp.float32)
                      for v in (a, b, d, f0, f1, f2, f3, f4)])

  spmm = _make_spmm(npad, H, e2, E, per0, per1)
  merge = _make_merge(npad, H, False)
  merge_aux = _make_merge(npad, H, True)

  p0, p1 = spmm(packed, xp)
  t1, out = merge(mk(c2, -1.0, 0.0, 0.0, w[0], w[1]), p0, p1, xp, xp, xp)
  tprev, tcur = xp, t1
  for k in range(1, K):
    p0, p1 = spmm(packed, tcur)
    if k < K - 1:
      coef = mk(2.0 * c2, -2.0, -1.0, 1.0, 0.0, w[k + 1])
      tnext, out = merge(coef, p0, p1, tcur, tprev, out)
    else:
      coef = mk(2.0 * c2, -2.0, -1.0, -1.0, 0.0, -w[k + 1],
                alpha * 0.5, -alpha * 0.5)
      tnext, out = merge_aux(coef, p0, p1, tcur, tprev, out, xp, t1)
    tprev, tcur = tcur, tnext
  return out[:N]


# speedup vs baseline: 1.0709x; 1.0709x over previous
import functools

import jax
import jax.numpy as jnp
from jax import lax
from jax.experimental import pallas as pl
from jax.experimental.pallas import tpu as pltpu
from jax.experimental.pallas import tpu_sc as plsc

G = 128
SUPER = 2
RC = 32
NTILE = 16
BR = 1024


def _edge_split(e2, etot, c, s):
  base_c = jnp.where(c == 0, 0, e2)
  cnt = jnp.where(c == 0, e2, etot - e2)
  per = ((cnt + NTILE - 1) // NTILE + 7) // 8 * 8
  my0 = base_c + s * per
  mycnt = jnp.clip(cnt - s * per, 0, per)
  return my0, mycnt


@functools.lru_cache(maxsize=None)
def _make_spmm(npad, H, e2, etot, per0, per1):
  PT = npad // NTILE
  NQ = H // 16
  CH = SUPER * G
  mesh = plsc.VectorSubcoreMesh(core_axis_name="c", subcore_axis_name="s")

  scratch = [
      pltpu.VMEM_SHARED((npad, H), jnp.float32),
      pltpu.VMEM((2 * 3 * G,), jnp.int32),
      pltpu.VMEM((2, G), jnp.int32),
      pltpu.VMEM((2, G), jnp.float32),
      pltpu.VMEM((2, G, H), jnp.float32),
      pltpu.VMEM((RC, H), jnp.float32),
      pltpu.SemaphoreType.DMA,
      pltpu.SemaphoreType.DMA,
      pltpu.SemaphoreType.DMA,
      pltpu.SemaphoreType.DMA,
      pltpu.SemaphoreType.DMA,
      pltpu.SemaphoreType.DMA,
      pltpu.SemaphoreType.DMA,
  ]

  def body(packed_h, u_h, p0_h, p1_h,
           acc, pbuf, rbuf, vbuf, gbuf, zbuf, semz, semi0, semi1, semg0,
           semg1, sems0, sems1):
    c = lax.axis_index("c")
    s = lax.axis_index("s")
    semi = [semi0, semi1]
    semg = [semg0, semg1]
    sems = [sems0, sems1]
    per = jnp.where(c == 0, per0, per1)
    my0 = jnp.where(c == 0, 0, e2) + s * per
    cnt = jnp.where(c == 0, e2, etot - e2)
    mycnt = jnp.clip(cnt - s * per, 0, per)
    nch = (mycnt + G - 1) // G

    def zrow(i, _):
      for q in range(NQ):
        zbuf[i, pl.ds(q * 16, 16)] = jnp.zeros((16,), jnp.float32)
      return 0
    lax.fori_loop(0, RC, zrow, 0)

    r0 = s * PT

    def zfire(i, _):
      ri = pl.multiple_of(r0 + i * RC, 8)
      pltpu.async_copy(zbuf, acc.at[pl.ds(ri, RC)], semz)
      return 0
    lax.fori_loop(0, PT // RC, zfire, 0)

    def zdrain(i, _):
      ri = pl.multiple_of(r0 + i * RC, 8)
      pltpu.make_async_copy(zbuf, acc.at[pl.ds(ri, RC)], semz).wait()
      return 0
    lax.fori_loop(0, PT // RC, zdrain, 0)
    plsc.subcore_barrier()

    lanes0 = lax.iota(jnp.int32, 16)

    def pslice(b, part, n=3 * G):
      return pbuf.at[pl.ds(b * 3 * G + part * G, n)]

    def idx_src(i):
      off3 = pl.multiple_of((my0 // G + i) * 3 * G, 8)
      return packed_h.at[pl.ds(off3, 3 * G)]

    def idx_issue(i, b):
      pltpu.async_copy(idx_src(i), pslice(b, 0), semi[b])

    def idx_wait(i, b):
      pltpu.make_async_copy(idx_src(i), pslice(b, 0), semi[b]).wait()

    def gather_issue(b):
      pltpu.async_copy(u_h.at[pslice(b, 0, G)], gbuf.at[b], semg[b])

    def gather_wait(b):
      pltpu.make_async_copy(u_h.at[pslice(b, 0, G)], gbuf.at[b],
                            semg[b]).wait()

    def stage(b):
      for j in range(8):
        sl = pl.ds(j * 16, 16)
        vbuf[b, sl] = lax.bitcast_convert_type(
            pbuf[pl.ds(b * 3 * G + G + j * 16, 16)], jnp.float32)
        rbuf[b, sl] = pbuf[pl.ds(b * 3 * G + 2 * G + j * 16, 16)]

    def process(i, b):
      rem = mycnt - i * G

      def grp(j, _):
        vv = vbuf[b, pl.ds(j * 16, 16)]
        vv = jnp.where(j * 16 + lanes0 < rem, vv, 0.0)
        for l in range(16):
          val = vv[l]
          g = j * 16 + l
          for qq in range(NQ):
            sl = pl.ds(qq * 16, 16)
            gbuf[b, g, sl] = gbuf[b, g, sl] * val
        return 0
      lax.fori_loop(0, G // 16, grp, 0)
      pltpu.async_copy(gbuf.at[b], acc.at[rbuf.at[b]], sems[b], add=True)

    def scat_wait(b):
      pltpu.make_async_copy(gbuf.at[b], acc.at[rbuf.at[b]],
                            sems[b]).wait()

    @pl.when(nch > 0)
    def _():
      pltpu.sync_copy(idx_src(0), pslice(0, 0))
      gather_issue(0)

    @pl.when(nch > 1)
    def _():
      idx_issue(1, 1)

    def pair(p, _):
      i0, i1, i2, i3 = 2 * p, 2 * p + 1, 2 * p + 2, 2 * p + 3

      @pl.when(i1 < nch)
      def _():
        idx_wait(i1, 1)
        gather_issue(1)

      gather_wait(0)
      stage(0)

      @pl.when(i2 < nch)
      def _():
        idx_issue(i2, 0)

      process(i0, 0)

      @pl.when(i1 < nch)
      def _():
        gather_wait(1)
        stage(1)

      @pl.when(i3 < nch)
      def _():
        idx_issue(i3, 1)

      @pl.when(i1 < nch)
      def _():
        process(i1, 1)

      scat_wait(0)

      @pl.when(i2 < nch)
      def _():
        idx_wait(i2, 0)
        gather_issue(0)

      @pl.when(i1 < nch)
      def _():
        scat_wait(1)
      return 0
    lax.fori_loop(0, (nch + 1) // 2, pair, 0)
    plsc.subcore_barrier()

    @pl.when(jnp.logical_and(s == 0, c == 0))
    def _():
      pltpu.sync_copy(acc, p0_h)

    @pl.when(jnp.logical_and(s == 0, c == 1))
    def _():
      pltpu.sync_copy(acc, p1_h)

  out_type = [jax.ShapeDtypeStruct((npad, H), jnp.float32)] * 2
  return pl.kernel(body, out_type=out_type, mesh=mesh,
                   scratch_types=scratch, name="cheb_spmm")


def _edge_pad_len(E, e2):
  per0 = -(-(-(-e2 // NTILE)) // G) * G
  per1 = -(-(-(-(E - e2) // NTILE)) // G) * G
  end0 = NTILE * per0 + SUPER * G
  end1 = e2 + NTILE * per1 + SUPER * G
  n = max(end0, end1, E)
  return -(-n // G) * G, per0, per1


@functools.lru_cache(maxsize=None)
def _make_deg(npad, ep, e2, etot):
  mesh = plsc.VectorSubcoreMesh(core_axis_name="c", subcore_axis_name="s")
  PT = npad // NTILE

  scratch = [
      pltpu.VMEM_SHARED((npad,), jnp.float32),
      pltpu.VMEM((G,), jnp.int32),
      pltpu.VMEM((G,), jnp.float32),
      pltpu.VMEM((RC * 16,), jnp.float32),
  ]

  def body(rows_h, d0_h, d1_h, accd, rbuf, obuf, zbuf):
    c = lax.axis_index("c")
    s = lax.axis_index("s")
    my0, mycnt = _edge_split(e2, etot, c, s)
    nch = (mycnt + G - 1) // G

    def zrow(i, _):
      zbuf[pl.ds(i * 16, 16)] = jnp.zeros((16,), jnp.float32)
      return 0
    lax.fori_loop(0, RC, zrow, 0)
    r0 = s * PT

    def zc(i, _):
      ri = pl.multiple_of(r0 + i * RC * 16, 8)
      pltpu.sync_copy(zbuf, accd.at[pl.ds(ri, RC * 16)])
      return 0
    lax.fori_loop(0, PT // (RC * 16), zc, 0)
    plsc.subcore_barrier()

    lanes0 = lax.iota(jnp.int32, 16)

    def chunk(i, _):
      off = pl.multiple_of(my0 + i * G, 8)
      pltpu.sync_copy(rows_h.at[pl.ds(off, G)], rbuf)
      rem = mycnt - i * G
      for j in range(G // 16):
        ones = jnp.where(j * 16 + lanes0 < rem, 1.0, 0.0)
        obuf[pl.ds(j * 16, 16)] = ones
      pltpu.sync_copy(obuf, accd.at[rbuf], add=True)
      return 0
    lax.fori_loop(0, nch, chunk, 0)
    plsc.subcore_barrier()

    @pl.when(jnp.logical_and(s == 0, c == 0))
    def _():
      pltpu.sync_copy(accd, d0_h)

    @pl.when(jnp.logical_and(s == 0, c == 1))
    def _():
      pltpu.sync_copy(accd, d1_h)

  out_type = [jax.ShapeDtypeStruct((npad,), jnp.float32)] * 2
  return pl.kernel(body, out_type=out_type, mesh=mesh,
                   scratch_types=scratch, name="deg_count")


def _degmax_tc(d0, d1):
  def body(a_ref, b_ref, o_ref):
    m = jnp.max(a_ref[...] + b_ref[...])
    o_ref[...] = jnp.full((8, 128), m, jnp.float32)
  return pl.pallas_call(
      body, out_shape=jax.ShapeDtypeStruct((8, 128), jnp.float32))(d0, d1)


@functools.lru_cache(maxsize=None)
def _make_merge(npad, H, aux):
  grid = npad // BR

  def body(*refs):
    if aux:
      coef, p0, p1, u, p, o, xa, ta, t_out, o_out = refs
    else:
      coef, p0, p1, u, p, o, t_out, o_out = refs
    a_, b_, d_ = coef[0], coef[1], coef[2]
    f0, f1, f2 = coef[3], coef[4], coef[5]
    uv = u[...]
    t = a_ * (p0[...] + p1[...]) + b_ * uv + d_ * p[...]
    oo = f0 * o[...] + f1 * uv + f2 * t
    if aux:
      oo = oo + coef[6] * xa[...] + coef[7] * ta[...]
    t_out[...] = t
    o_out[...] = oo

  narr = 7 if aux else 5
  bspec = pl.BlockSpec((BR, H), lambda i: (i, 0))
  return pl.pallas_call(
      body,
      grid=(grid,),
      in_specs=[pl.BlockSpec(memory_space=pltpu.SMEM)] + [bspec] * narr,
      out_specs=[bspec, bspec],
      out_shape=[jax.ShapeDtypeStruct((npad, H), jnp.float32)] * 2,
  )


def kernel(x, edge_index, edge_vals, poly_logits, hp_alpha):
  N, H = x.shape
  E = edge_index.shape[1]
  K = poly_logits.shape[0] - 1
  npad = -(-N // BR) * BR

  rows = edge_index[0]
  cols = edge_index[1]
  e2 = -(-(-(-E // 2)) // G) * G
  nbg, per0, per1 = _edge_pad_len(E, e2)
  zpad = jnp.zeros((nbg - E,), jnp.int32)
  rows_p = jnp.concatenate([rows, zpad])
  cols_p = jnp.concatenate([cols, zpad])
  vbits = lax.bitcast_convert_type(edge_vals, jnp.int32)
  vals_p = jnp.concatenate([vbits, zpad])
  nb = nbg // G
  packed = jnp.stack([cols_p.reshape(nb, G), vals_p.reshape(nb, G),
                      rows_p.reshape(nb, G)], axis=1).reshape(-1)
  ep = nbg

  xp = jnp.concatenate([x, jnp.zeros((npad - N, H), jnp.float32)])

  deg = _make_deg(npad, ep, e2, E)
  d0, d1 = deg(rows_p)
  dm = _degmax_tc(d0.reshape(npad // 128, 128), d1.reshape(npad // 128, 128))
  lam = 2.0 * dm[0, 0]
  c2 = 2.0 / lam

  w = jax.nn.softmax(poly_logits)
  alpha = hp_alpha.astype(jnp.float32)

  def mk(a, b, d, f0, f1, f2, f3=0.0, f4=0.0):
    return jnp.stack([jnp.asarray(v, jnp.float32) * jnp.ones((), jnp.float32)
                      for v in (a, b, d, f0, f1, f2, f3, f4)])

  spmm = _make_spmm(npad, H, e2, E, per0, per1)
  merge = _make_merge(npad, H, False)
  merge_aux = _make_merge(npad, H, True)

  p0, p1 = spmm(packed, xp)
  t1, out = merge(mk(c2, -1.0, 0.0, 0.0, w[0], w[1]), p0, p1, xp, xp, xp)
  tprev, tcur = xp, t1
  for k in range(1, K):
    p0, p1 = spmm(packed, tcur)
    if k < K - 1:
      coef = mk(2.0 * c2, -2.0, -1.0, 1.0, 0.0, w[k + 1])
      tnext, out = merge(coef, p0, p1, tcur, tprev, out)
    else:
      coef = mk(2.0 * c2, -2.0, -1.0, -1.0, 0.0, -w[k + 1],
                alpha * 0.5, -alpha * 0.5)
      tnext, out = merge_aux(coef, p0, p1, tcur, tprev, out, xp, t1)
    tprev, tcur = tcur, tnext
  return out[:N]
